# baseline (speedup 1.0000x reference)
import jax
import jax.numpy as jnp
from jax import lax
from jax.experimental import pallas as pl
from jax.experimental.pallas import tpu as pltpu
from jax.experimental.pallas import tpu_sc as plsc

BATCH = 1024
CONTEXT = 20
EMB_DIM = 64
VOCAB = 100000

VT = 2048
NV = -(-VOCAB // VT)
VPAD = NV * VT
NEG = -1e30

NUM_WORKERS = 32
N_IDX = BATCH * CONTEXT
B_PER_W = N_IDX // NUM_WORKERS


def _sc_gather_body(table_hbm, idx_hbm, out_hbm, idx_v, rows_v, sem):
    wid = lax.axis_index("s") * 2 + lax.axis_index("c")
    base = wid * B_PER_W
    pltpu.sync_copy(idx_hbm.at[pl.ds(base, B_PER_W)], idx_v)
    pltpu.async_copy(table_hbm.at[idx_v], rows_v, sem).wait()
    pltpu.sync_copy(rows_v, out_hbm.at[pl.ds(base, B_PER_W)])


def _sc_gather(table, idx):
    mesh = plsc.VectorSubcoreMesh(core_axis_name="c", subcore_axis_name="s")
    k = pl.kernel(
        _sc_gather_body,
        out_type=jax.ShapeDtypeStruct((N_IDX, EMB_DIM), jnp.float32),
        mesh=mesh,
        compiler_params=pltpu.CompilerParams(use_tc_tiling_on_sc=False),
        scratch_types=[
            pltpu.VMEM((B_PER_W,), jnp.int32),
            pltpu.VMEM((B_PER_W, EMB_DIM), jnp.float32),
            pltpu.SemaphoreType.DMA,
        ],
    )
    return k(table, idx)


def _stats_body(g_ref, w_ref, b_ref, xsum_ref, lse_ref, m_ref):
    j = pl.program_id(0)

    @pl.when(j == 0)
    def _init():
        acc = g_ref[pl.ds(0, BATCH), :]
        for c in range(1, CONTEXT):
            acc = acc + g_ref[pl.ds(c * BATCH, BATCH), :]
        xsum_ref[...] = acc
        m_ref[...] = jnp.full((BATCH, 1), -jnp.inf, jnp.float32)
        lse_ref[...] = jnp.zeros((BATCH, 1), jnp.float32)

    x = xsum_ref[...]
    logits = lax.dot_general(
        x, w_ref[...], (((1,), (1,)), ((), ())),
        preferred_element_type=jnp.float32,
        precision=lax.Precision.HIGHEST,
    ) + b_ref[0, :][None, :]
    tmax = jnp.max(logits, axis=1, keepdims=True)
    m_old = m_ref[...]
    m_new = jnp.maximum(m_old, tmax)
    s = lse_ref[...] * jnp.exp(m_old - m_new)
    s = s + jnp.sum(jnp.exp(logits - m_new), axis=1, keepdims=True)
    lse_ref[...] = s
    m_ref[...] = m_new

    @pl.when(j == NV - 1)
    def _fin():
        lse_ref[...] = m_ref[...] + jnp.log(lse_ref[...])


def _stats(gathered, w_pad, b_pad):
    return pl.pallas_call(
        _stats_body,
        grid=(NV,),
        in_specs=[
            pl.BlockSpec((N_IDX, EMB_DIM), lambda j: (0, 0)),
            pl.BlockSpec((VT, EMB_DIM), lambda j: (j, 0)),
            pl.BlockSpec((1, VT), lambda j: (0, j)),
        ],
        out_specs=[
            pl.BlockSpec((BATCH, EMB_DIM), lambda j: (0, 0)),
            pl.BlockSpec((BATCH, 1), lambda j: (0, 0)),
            pl.BlockSpec((BATCH, 1), lambda j: (0, 0)),
        ],
        out_shape=[
            jax.ShapeDtypeStruct((BATCH, EMB_DIM), jnp.float32),
            jax.ShapeDtypeStruct((BATCH, 1), jnp.float32),
            jax.ShapeDtypeStruct((BATCH, 1), jnp.float32),
        ],
    )(gathered, w_pad, b_pad)


def _out_body(x_ref, w_ref, b_ref, lse_ref, o_ref):
    logits = lax.dot_general(
        x_ref[...], w_ref[...], (((1,), (1,)), ((), ())),
        preferred_element_type=jnp.float32,
        precision=lax.Precision.HIGHEST,
    ) + b_ref[0, :][None, :]
    o_ref[...] = logits - lse_ref[...]


def _final(xsum, w_pad, b_pad, lse):
    return pl.pallas_call(
        _out_body,
        grid=(NV,),
        in_specs=[
            pl.BlockSpec((BATCH, EMB_DIM), lambda j: (0, 0)),
            pl.BlockSpec((VT, EMB_DIM), lambda j: (j, 0)),
            pl.BlockSpec((1, VT), lambda j: (0, j)),
            pl.BlockSpec((BATCH, 1), lambda j: (0, 0)),
        ],
        out_specs=pl.BlockSpec((BATCH, VT), lambda j: (0, j)),
        out_shape=jax.ShapeDtypeStruct((BATCH, VOCAB), jnp.float32),
    )(xsum, w_pad, b_pad, lse)


def kernel(inputs, emb_table, W, b):
    idx = inputs.astype(jnp.int32).T.reshape(-1)
    gathered = _sc_gather(emb_table, idx)

    w_pad = jnp.pad(W, ((0, VPAD - VOCAB), (0, 0)))
    b_pad = jnp.pad(b, (0, VPAD - VOCAB), constant_values=NEG).reshape(1, VPAD)

    xsum, lse, _ = _stats(gathered, w_pad, b_pad)
    return _final(xsum, w_pad, b_pad, lse)


# speedup vs baseline: 1.6115x; 1.6115x over previous
import jax
import jax.numpy as jnp
from jax import lax
from jax.experimental import pallas as pl
from jax.experimental.pallas import tpu as pltpu
from jax.experimental.pallas import tpu_sc as plsc

BATCH = 1024
CONTEXT = 20
EMB_DIM = 64
VOCAB = 100000

VT = 2048
NV = -(-VOCAB // VT)
VPAD = NV * VT
NEG = -1e30

NUM_WORKERS = 32
N_IDX = BATCH * CONTEXT
B_PER_W = N_IDX // NUM_WORKERS


def _sc_gather_body(table_hbm, idx_hbm, out_hbm, idx_v, rows_v, sem):
    wid = lax.axis_index("s") * 2 + lax.axis_index("c")
    base = wid * B_PER_W
    pltpu.sync_copy(idx_hbm.at[pl.ds(base, B_PER_W)], idx_v)
    pltpu.async_copy(table_hbm.at[idx_v], rows_v, sem).wait()
    pltpu.sync_copy(rows_v, out_hbm.at[pl.ds(base, B_PER_W)])


def _sc_gather(table, idx):
    mesh = plsc.VectorSubcoreMesh(core_axis_name="c", subcore_axis_name="s")
    k = pl.kernel(
        _sc_gather_body,
        out_type=jax.ShapeDtypeStruct((N_IDX, EMB_DIM), jnp.float32),
        mesh=mesh,
        compiler_params=pltpu.CompilerParams(use_tc_tiling_on_sc=False),
        scratch_types=[
            pltpu.VMEM((B_PER_W,), jnp.int32),
            pltpu.VMEM((B_PER_W, EMB_DIM), jnp.float32),
            pltpu.SemaphoreType.DMA,
        ],
    )
    return k(table, idx)


def _stats_body(g_ref, w_ref, b_ref, xsum_ref, lse_ref, m_ref):
    j = pl.program_id(0)

    @pl.when(j == 0)
    def _init():
        acc = g_ref[pl.ds(0, BATCH), :]
        for c in range(1, CONTEXT):
            acc = acc + g_ref[pl.ds(c * BATCH, BATCH), :]
        xsum_ref[...] = acc
        m_ref[...] = jnp.full((BATCH, 1), -jnp.inf, jnp.float32)
        lse_ref[...] = jnp.zeros((BATCH, 1), jnp.float32)

    x = xsum_ref[...].astype(jnp.bfloat16)
    logits = lax.dot_general(
        x, w_ref[...], (((1,), (1,)), ((), ())),
        preferred_element_type=jnp.float32,
    ) + b_ref[0, :][None, :]
    tmax = jnp.max(logits, axis=1, keepdims=True)
    m_old = m_ref[...]
    m_new = jnp.maximum(m_old, tmax)
    s = lse_ref[...] * jnp.exp(m_old - m_new)
    s = s + jnp.sum(jnp.exp(logits - m_new), axis=1, keepdims=True)
    lse_ref[...] = s
    m_ref[...] = m_new

    @pl.when(j == NV - 1)
    def _fin():
        lse_ref[...] = m_ref[...] + jnp.log(lse_ref[...])


def _stats(gathered, w_pad, b_pad):
    return pl.pallas_call(
        _stats_body,
        grid=(NV,),
        in_specs=[
            pl.BlockSpec((N_IDX, EMB_DIM), lambda j: (0, 0)),
            pl.BlockSpec((VT, EMB_DIM), lambda j: (j, 0)),
            pl.BlockSpec((1, VT), lambda j: (0, j)),
        ],
        out_specs=[
            pl.BlockSpec((BATCH, EMB_DIM), lambda j: (0, 0)),
            pl.BlockSpec((BATCH, 1), lambda j: (0, 0)),
            pl.BlockSpec((BATCH, 1), lambda j: (0, 0)),
        ],
        out_shape=[
            jax.ShapeDtypeStruct((BATCH, EMB_DIM), jnp.float32),
            jax.ShapeDtypeStruct((BATCH, 1), jnp.float32),
            jax.ShapeDtypeStruct((BATCH, 1), jnp.float32),
        ],
    )(gathered, w_pad, b_pad)


def _out_body(x_ref, w_ref, b_ref, lse_ref, o_ref):
    logits = lax.dot_general(
        x_ref[...].astype(jnp.bfloat16), w_ref[...], (((1,), (1,)), ((), ())),
        preferred_element_type=jnp.float32,
    ) + b_ref[0, :][None, :]
    o_ref[...] = logits - lse_ref[...]


def _final(xsum, w_pad, b_pad, lse):
    return pl.pallas_call(
        _out_body,
        grid=(NV,),
        in_specs=[
            pl.BlockSpec((BATCH, EMB_DIM), lambda j: (0, 0)),
            pl.BlockSpec((VT, EMB_DIM), lambda j: (j, 0)),
            pl.BlockSpec((1, VT), lambda j: (0, j)),
            pl.BlockSpec((BATCH, 1), lambda j: (0, 0)),
        ],
        out_specs=pl.BlockSpec((BATCH, VT), lambda j: (0, j)),
        out_shape=jax.ShapeDtypeStruct((BATCH, VOCAB), jnp.float32),
    )(xsum, w_pad, b_pad, lse)


def kernel(inputs, emb_table, W, b):
    idx = inputs.astype(jnp.int32).T.reshape(-1)
    gathered = _sc_gather(emb_table, idx)

    w_pad = jnp.pad(W, ((0, VPAD - VOCAB), (0, 0))).astype(jnp.bfloat16)
    b_pad = jnp.pad(b, (0, VPAD - VOCAB), constant_values=NEG).reshape(1, VPAD)

    xsum, lse, _ = _stats(gathered, w_pad, b_pad)
    return _final(xsum, w_pad, b_pad, lse)


# speedup vs baseline: 1.6415x; 1.0186x over previous
import jax
import jax.numpy as jnp
from jax import lax
from jax.experimental import pallas as pl
from jax.experimental.pallas import tpu as pltpu
from jax.experimental.pallas import tpu_sc as plsc

BATCH = 1024
CONTEXT = 20
EMB_DIM = 64
VOCAB = 100000

VT = 2048
NV = -(-VOCAB // VT)
VPAD = NV * VT
NEG = -1e30

NUM_WORKERS = 32
N_IDX = BATCH * CONTEXT
B_PER_W = N_IDX // NUM_WORKERS
ROWS_PER_W = BATCH // NUM_WORKERS
LCHUNK = 16


def _sc_gather_sum_body(table_hbm, idx_hbm, out_hbm, idx_v, rows_v, acc_v, sem):
    wid = lax.axis_index("s") * 2 + lax.axis_index("c")
    base = wid * B_PER_W
    pltpu.sync_copy(idx_hbm.at[pl.ds(base, B_PER_W)], idx_v)
    pltpu.async_copy(table_hbm.at[idx_v], rows_v, sem).wait()

    @pl.loop(0, ROWS_PER_W)
    def _(r):
        rbase = r * CONTEXT
        for k in range(EMB_DIM // LCHUNK):
            sl = pl.ds(k * LCHUNK, LCHUNK)
            acc = rows_v[rbase, sl]
            for c in range(1, CONTEXT):
                acc = acc + rows_v[rbase + c, sl]
            acc_v[r, sl] = acc

    pltpu.sync_copy(acc_v, out_hbm.at[pl.ds(wid * ROWS_PER_W, ROWS_PER_W)])


def _sc_gather_sum(table, idx):
    mesh = plsc.VectorSubcoreMesh(core_axis_name="c", subcore_axis_name="s")
    k = pl.kernel(
        _sc_gather_sum_body,
        out_type=jax.ShapeDtypeStruct((BATCH, EMB_DIM), jnp.float32),
        mesh=mesh,
        compiler_params=pltpu.CompilerParams(use_tc_tiling_on_sc=False),
        scratch_types=[
            pltpu.VMEM((B_PER_W,), jnp.int32),
            pltpu.VMEM((B_PER_W, EMB_DIM), jnp.float32),
            pltpu.VMEM((ROWS_PER_W, EMB_DIM), jnp.float32),
            pltpu.SemaphoreType.DMA,
        ],
    )
    return k(table, idx)


def _stats_body(x_ref, w_ref, b_ref, lse_ref, m_ref):
    j = pl.program_id(0)

    @pl.when(j == 0)
    def _init():
        m_ref[...] = jnp.full((BATCH, 1), -jnp.inf, jnp.float32)
        lse_ref[...] = jnp.zeros((BATCH, 1), jnp.float32)

    x = x_ref[...].astype(jnp.bfloat16)
    logits = lax.dot_general(
        x, w_ref[...], (((1,), (1,)), ((), ())),
        preferred_element_type=jnp.float32,
    ) + b_ref[0, :][None, :]
    tmax = jnp.max(logits, axis=1, keepdims=True)
    m_old = m_ref[...]
    m_new = jnp.maximum(m_old, tmax)
    s = lse_ref[...] * jnp.exp(m_old - m_new)
    s = s + jnp.sum(jnp.exp(logits - m_new), axis=1, keepdims=True)
    lse_ref[...] = s
    m_ref[...] = m_new

    @pl.when(j == NV - 1)
    def _fin():
        lse_ref[...] = m_ref[...] + jnp.log(lse_ref[...])


def _stats(xsum, w_pad, b_pad):
    return pl.pallas_call(
        _stats_body,
        grid=(NV,),
        in_specs=[
            pl.BlockSpec((BATCH, EMB_DIM), lambda j: (0, 0)),
            pl.BlockSpec((VT, EMB_DIM), lambda j: (j, 0)),
            pl.BlockSpec((1, VT), lambda j: (0, j)),
        ],
        out_specs=[
            pl.BlockSpec((BATCH, 1), lambda j: (0, 0)),
            pl.BlockSpec((BATCH, 1), lambda j: (0, 0)),
        ],
        out_shape=[
            jax.ShapeDtypeStruct((BATCH, 1), jnp.float32),
            jax.ShapeDtypeStruct((BATCH, 1), jnp.float32),
        ],
    )(xsum, w_pad, b_pad)


def _out_body(x_ref, w_ref, b_ref, lse_ref, o_ref):
    logits = lax.dot_general(
        x_ref[...].astype(jnp.bfloat16), w_ref[...], (((1,), (1,)), ((), ())),
        preferred_element_type=jnp.float32,
    ) + b_ref[0, :][None, :]
    o_ref[...] = logits - lse_ref[...]


def _final(xsum, w_pad, b_pad, lse):
    return pl.pallas_call(
        _out_body,
        grid=(NV,),
        in_specs=[
            pl.BlockSpec((BATCH, EMB_DIM), lambda j: (0, 0)),
            pl.BlockSpec((VT, EMB_DIM), lambda j: (j, 0)),
            pl.BlockSpec((1, VT), lambda j: (0, j)),
            pl.BlockSpec((BATCH, 1), lambda j: (0, 0)),
        ],
        out_specs=pl.BlockSpec((BATCH, VT), lambda j: (0, j)),
        out_shape=jax.ShapeDtypeStruct((BATCH, VOCAB), jnp.float32),
    )(xsum, w_pad, b_pad, lse)


def kernel(inputs, emb_table, W, b):
    idx = inputs.astype(jnp.int32).reshape(-1)
    xsum = _sc_gather_sum(emb_table, idx)

    w_pad = jnp.pad(W, ((0, VPAD - VOCAB), (0, 0))).astype(jnp.bfloat16)
    b_pad = jnp.pad(b, (0, VPAD - VOCAB), constant_values=NEG).reshape(1, VPAD)

    lse, _ = _stats(xsum, w_pad, b_pad)
    return _final(xsum, w_pad, b_pad, lse)


# speedup vs baseline: 1.6552x; 1.0084x over previous
import jax
import jax.numpy as jnp
from jax import lax
from jax.experimental import pallas as pl
from jax.experimental.pallas import tpu as pltpu
from jax.experimental.pallas import tpu_sc as plsc

BATCH = 1024
CONTEXT = 20
EMB_DIM = 64
VOCAB = 100000

VT = 2176
NV = -(-VOCAB // VT)
VPAD = NV * VT
NEG = -1e30

NUM_WORKERS = 32
N_IDX = BATCH * CONTEXT
B_PER_W = N_IDX // NUM_WORKERS
ROWS_PER_W = BATCH // NUM_WORKERS
LCHUNK = 16


def _sc_gather_sum_body(table_hbm, idx_hbm, out_hbm, idx_v, rows_v, acc_v, sem):
    wid = lax.axis_index("s") * 2 + lax.axis_index("c")
    base = wid * B_PER_W
    pltpu.sync_copy(idx_hbm.at[pl.ds(base, B_PER_W)], idx_v)
    pltpu.async_copy(table_hbm.at[idx_v], rows_v, sem).wait()

    @pl.loop(0, ROWS_PER_W)
    def _(r):
        rbase = r * CONTEXT
        for k in range(EMB_DIM // LCHUNK):
            sl = pl.ds(k * LCHUNK, LCHUNK)
            acc = rows_v[rbase, sl]
            for c in range(1, CONTEXT):
                acc = acc + rows_v[rbase + c, sl]
            acc_v[r, sl] = acc

    pltpu.sync_copy(acc_v, out_hbm.at[pl.ds(wid * ROWS_PER_W, ROWS_PER_W)])


def _sc_gather_sum(table, idx):
    mesh = plsc.VectorSubcoreMesh(core_axis_name="c", subcore_axis_name="s")
    k = pl.kernel(
        _sc_gather_sum_body,
        out_type=jax.ShapeDtypeStruct((BATCH, EMB_DIM), jnp.float32),
        mesh=mesh,
        compiler_params=pltpu.CompilerParams(use_tc_tiling_on_sc=False),
        scratch_types=[
            pltpu.VMEM((B_PER_W,), jnp.int32),
            pltpu.VMEM((B_PER_W, EMB_DIM), jnp.float32),
            pltpu.VMEM((ROWS_PER_W, EMB_DIM), jnp.float32),
            pltpu.SemaphoreType.DMA,
        ],
    )
    return k(table, idx)


def _stats_body(x_ref, w_ref, b_ref, lse_ref, m_ref):
    j = pl.program_id(0)

    @pl.when(j == 0)
    def _init():
        m_ref[...] = jnp.full((BATCH, 1), -jnp.inf, jnp.float32)
        lse_ref[...] = jnp.zeros((BATCH, 1), jnp.float32)

    x = x_ref[...].astype(jnp.bfloat16)
    logits = lax.dot_general(
        x, w_ref[...], (((1,), (1,)), ((), ())),
        preferred_element_type=jnp.float32,
    ) + b_ref[0, :][None, :]
    tmax = jnp.max(logits, axis=1, keepdims=True)
    m_old = m_ref[...]
    m_new = jnp.maximum(m_old, tmax)
    s = lse_ref[...] * jnp.exp(m_old - m_new)
    s = s + jnp.sum(jnp.exp(logits - m_new), axis=1, keepdims=True)
    lse_ref[...] = s
    m_ref[...] = m_new

    @pl.when(j == NV - 1)
    def _fin():
        lse_ref[...] = m_ref[...] + jnp.log(lse_ref[...])


def _stats(xsum, w_pad, b_pad):
    return pl.pallas_call(
        _stats_body,
        grid=(NV,),
        in_specs=[
            pl.BlockSpec((BATCH, EMB_DIM), lambda j: (0, 0)),
            pl.BlockSpec((VT, EMB_DIM), lambda j: (j, 0)),
            pl.BlockSpec((1, VT), lambda j: (0, j)),
        ],
        out_specs=[
            pl.BlockSpec((BATCH, 1), lambda j: (0, 0)),
            pl.BlockSpec((BATCH, 1), lambda j: (0, 0)),
        ],
        out_shape=[
            jax.ShapeDtypeStruct((BATCH, 1), jnp.float32),
            jax.ShapeDtypeStruct((BATCH, 1), jnp.float32),
        ],
    )(xsum, w_pad, b_pad)


def _out_body(x_ref, w_ref, b_ref, lse_ref, o_ref):
    logits = lax.dot_general(
        x_ref[...].astype(jnp.bfloat16), w_ref[...], (((1,), (1,)), ((), ())),
        preferred_element_type=jnp.float32,
    ) + b_ref[0, :][None, :]
    o_ref[...] = logits - lse_ref[...]


def _final(xsum, w_pad, b_pad, lse):
    return pl.pallas_call(
        _out_body,
        grid=(NV,),
        in_specs=[
            pl.BlockSpec((BATCH, EMB_DIM), lambda j: (0, 0)),
            pl.BlockSpec((VT, EMB_DIM), lambda j: (j, 0)),
            pl.BlockSpec((1, VT), lambda j: (0, j)),
            pl.BlockSpec((BATCH, 1), lambda j: (0, 0)),
        ],
        out_specs=pl.BlockSpec((BATCH, VT), lambda j: (0, j)),
        out_shape=jax.ShapeDtypeStruct((BATCH, VOCAB), jnp.float32),
    )(xsum, w_pad, b_pad, lse)


def kernel(inputs, emb_table, W, b):
    idx = inputs.astype(jnp.int32).reshape(-1)
    xsum = _sc_gather_sum(emb_table, idx)

    w_pad = jnp.pad(W, ((0, VPAD - VOCAB), (0, 0))).astype(jnp.bfloat16)
    b_pad = jnp.pad(b, (0, VPAD - VOCAB), constant_values=NEG).reshape(1, VPAD)

    lse, _ = _stats(xsum, w_pad, b_pad)
    return _final(xsum, w_pad, b_pad, lse)


# speedup vs baseline: 2.8515x; 1.7227x over previous
import jax
import jax.numpy as jnp
from jax import lax
from jax.experimental import pallas as pl
from jax.experimental.pallas import tpu as pltpu
from jax.experimental.pallas import tpu_sc as plsc

BATCH = 1024
CONTEXT = 20
EMB_DIM = 64
AUG = EMB_DIM + 1
VOCAB = 100000

VT = 2176
NV = -(-VOCAB // VT)
VPAD = NV * VT
NEG = -1e30

NUM_WORKERS = 32
N_IDX = BATCH * CONTEXT
B_PER_W = N_IDX // NUM_WORKERS
ROWS_PER_W = BATCH // NUM_WORKERS
LCHUNK = 16


def _sc_gather_sum_body(table_hbm, idx_hbm, out_hbm, idx_v, rows_v, acc_v, sem):
    wid = lax.axis_index("s") * 2 + lax.axis_index("c")
    col0 = wid * ROWS_PER_W
    for c in range(CONTEXT):
        pltpu.sync_copy(
            idx_hbm.at[pl.ds(c * BATCH + col0, ROWS_PER_W)],
            idx_v.at[pl.ds(c * ROWS_PER_W, ROWS_PER_W)],
        )
    pltpu.async_copy(table_hbm.at[idx_v], rows_v, sem).wait()

    @pl.loop(0, ROWS_PER_W)
    def _(r):
        for k in range(EMB_DIM // LCHUNK):
            sl = pl.ds(k * LCHUNK, LCHUNK)
            acc = rows_v[r, sl]
            for c in range(1, CONTEXT):
                acc = acc + rows_v[c * ROWS_PER_W + r, sl]
            acc_v[r, sl] = acc

    pltpu.sync_copy(acc_v, out_hbm.at[pl.ds(col0, ROWS_PER_W)])


def _sc_gather_sum(table, idx):
    mesh = plsc.VectorSubcoreMesh(core_axis_name="c", subcore_axis_name="s")
    k = pl.kernel(
        _sc_gather_sum_body,
        out_type=jax.ShapeDtypeStruct((BATCH, EMB_DIM), jnp.float32),
        mesh=mesh,
        compiler_params=pltpu.CompilerParams(use_tc_tiling_on_sc=False),
        scratch_types=[
            pltpu.VMEM((B_PER_W,), jnp.int32),
            pltpu.VMEM((B_PER_W, EMB_DIM), jnp.float32),
            pltpu.VMEM((ROWS_PER_W, EMB_DIM), jnp.float32),
            pltpu.SemaphoreType.DMA,
        ],
    )
    return k(table, idx)


def _stats_body(x_ref, w_ref, lse_ref, m_ref):
    j = pl.program_id(0)

    @pl.when(j == 0)
    def _init():
        m_ref[...] = jnp.full((1, BATCH), -jnp.inf, jnp.float32)
        lse_ref[...] = jnp.zeros((1, BATCH), jnp.float32)

    lt = lax.dot_general(
        w_ref[...], x_ref[...].astype(jnp.bfloat16), (((1,), (0,)), ((), ())),
        preferred_element_type=jnp.float32,
    )
    tmax = jnp.max(lt, axis=0, keepdims=True)
    m_old = m_ref[...]
    m_new = jnp.maximum(m_old, tmax)
    s = lse_ref[...] * jnp.exp(m_old - m_new)
    s = s + jnp.sum(jnp.exp(lt - m_new), axis=0, keepdims=True)
    lse_ref[...] = s
    m_ref[...] = m_new

    @pl.when(j == NV - 1)
    def _fin():
        lse_ref[...] = m_ref[...] + jnp.log(lse_ref[...])


def _stats(x_aug_t, w_aug):
    return pl.pallas_call(
        _stats_body,
        grid=(NV,),
        in_specs=[
            pl.BlockSpec((AUG, BATCH), lambda j: (0, 0)),
            pl.BlockSpec((VT, AUG), lambda j: (j, 0)),
        ],
        out_specs=[
            pl.BlockSpec((1, BATCH), lambda j: (0, 0)),
            pl.BlockSpec((1, BATCH), lambda j: (0, 0)),
        ],
        out_shape=[
            jax.ShapeDtypeStruct((1, BATCH), jnp.float32),
            jax.ShapeDtypeStruct((1, BATCH), jnp.float32),
        ],
    )(x_aug_t, w_aug)


def _out_body(x_ref, w_ref, lse_ref, o_ref):
    lt = lax.dot_general(
        w_ref[...], x_ref[...].astype(jnp.bfloat16), (((1,), (0,)), ((), ())),
        preferred_element_type=jnp.float32,
    )
    o_ref[...] = lt - lse_ref[...]


def _final(x_aug_t, w_aug, lse):
    return pl.pallas_call(
        _out_body,
        grid=(NV,),
        in_specs=[
            pl.BlockSpec((AUG, BATCH), lambda j: (0, 0)),
            pl.BlockSpec((VT, AUG), lambda j: (j, 0)),
            pl.BlockSpec((1, BATCH), lambda j: (0, 0)),
        ],
        out_specs=pl.BlockSpec((VT, BATCH), lambda j: (j, 0)),
        out_shape=jax.ShapeDtypeStruct((VOCAB, BATCH), jnp.float32),
    )(x_aug_t, w_aug, lse)


def kernel(inputs, emb_table, W, b):
    idx = inputs.astype(jnp.int32).T.reshape(-1)
    xsum = _sc_gather_sum(emb_table, idx)

    x_aug_t = jnp.concatenate(
        [xsum, jnp.ones((BATCH, 1), jnp.float32)], axis=1).T
    w_aug = jnp.concatenate(
        [jnp.pad(W, ((0, VPAD - VOCAB), (0, 0))),
         jnp.pad(b, (0, VPAD - VOCAB), constant_values=NEG)[:, None]],
        axis=1).astype(jnp.bfloat16)

    lse, _ = _stats(x_aug_t, w_aug)
    out_t = _final(x_aug_t, w_aug, lse)
    return out_t.T
